# baseline (speedup 1.0000x reference)
import functools

import jax
import jax.numpy as jnp
from jax import lax
from jax.experimental import pallas as pl
from jax.experimental.pallas import tpu as pltpu
from jax.experimental.pallas import tpu_sc as plsc


def _make_gather(n_idx: int, chunk: int):
    info = plsc.get_sparse_core_info()
    nc, ns, nl = info.num_cores, info.num_subcores, info.num_lanes
    nw = nc * ns
    assert n_idx % nw == 0
    per_w = n_idx // nw
    assert per_w % chunk == 0 and chunk % (8 * nl) == 0
    n_chunks = per_w // chunk
    assert n_chunks % 2 == 0 and n_chunks >= 4

    mesh = plsc.VectorSubcoreMesh(core_axis_name="c", subcore_axis_name="s")

    @functools.partial(
        pl.kernel,
        mesh=mesh,
        out_type=jax.ShapeDtypeStruct((2 * n_idx, 16), jnp.float32),
        compiler_params=pltpu.CompilerParams(
            use_tc_tiling_on_sc=False, needs_layout_passes=False),
        scratch_types=[
            pltpu.VMEM((per_w,), jnp.int32),
            pltpu.VMEM((2 * chunk,), jnp.int32),
            pltpu.VMEM((2 * chunk,), jnp.int32),
            pltpu.VMEM((2 * chunk, 16), jnp.float32),
            pltpu.VMEM((2 * chunk, 16), jnp.float32),
            pltpu.SemaphoreType.DMA,
            pltpu.SemaphoreType.DMA,
            pltpu.SemaphoreType.DMA,
            pltpu.SemaphoreType.DMA,
        ],
    )
    def gather_kernel(table_hbm, idx_hbm, out_hbm, idx_v, q0, q1, r0, r1,
                      sg0, sg1, so0, so1):
        wid = lax.axis_index("s") * nc + lax.axis_index("c")
        base = wid * per_w
        pltpu.sync_copy(idx_hbm.at[pl.ds(base, per_w)], idx_v)
        lanes = lax.iota(jnp.int32, nl)

        def build_qidx(c, qbuf):
            def one(i, carry):
                v = idx_v[pl.ds(c * chunk + i * nl, nl)]
                two = v + v
                pos = 2 * nl * i + 2 * lanes
                plsc.store_scatter(qbuf, [pos], two)
                plsc.store_scatter(qbuf, [pos + 1], two + 1)
                return carry
            lax.fori_loop(0, chunk // nl, one, 0, unroll=4)

        def fire_gather(qbuf, buf, sem):
            pltpu.async_copy(table_hbm.at[qbuf], buf, sem)

        def wait_gather(buf, sem):
            pltpu.make_async_copy(table_hbm.at[q0], buf, sem).wait()

        def fire_out(c, buf, sem):
            pltpu.async_copy(
                buf, out_hbm.at[pl.ds(2 * (base + c * chunk), 2 * chunk)], sem)

        def wait_out(buf, sem):
            pltpu.make_async_copy(buf, out_hbm.at[pl.ds(0, 2 * chunk)],
                                  sem).wait()

        build_qidx(0, q0)
        fire_gather(q0, r0, sg0)
        build_qidx(1, q1)
        fire_gather(q1, r1, sg1)

        def step(i, carry):
            g = 2 * i
            wait_gather(r0, sg0)
            fire_out(g, r0, so0)
            build_qidx(g + 2, q0)
            wait_gather(r1, sg1)
            fire_out(g + 1, r1, so1)
            build_qidx(g + 3, q1)
            wait_out(r0, so0)
            fire_gather(q0, r0, sg0)
            wait_out(r1, so1)
            fire_gather(q1, r1, sg1)
            return carry

        lax.fori_loop(0, n_chunks // 2 - 1, step, 0)

        wait_gather(r0, sg0)
        fire_out(n_chunks - 2, r0, so0)
        wait_gather(r1, sg1)
        fire_out(n_chunks - 1, r1, so1)
        wait_out(r0, so0)
        wait_out(r1, so1)

    return gather_kernel


def kernel(input_seq, weight):
    b, s = input_seq.shape
    vocab, d = weight.shape
    assert d == 32
    idx = input_seq.reshape(-1).astype(jnp.int32)
    table = weight.reshape(vocab * 2, 16)
    out = _make_gather(b * s, chunk=1280)(table, idx)
    return out.reshape(b, s, d)
